# baseline (speedup 1.0000x reference)
import functools

import jax
import jax.numpy as jnp
from jax import lax
from jax.experimental import pallas as pl
from jax.experimental.pallas import tpu as pltpu
from jax.experimental.pallas import tpu_sc as plsc

N_ENTITIES = 14541
EMBED_DIM = 128
BATCH = 16384

NUM_CORES = 2
NUM_SUBCORES = 16
N_WORKERS = NUM_CORES * NUM_SUBCORES
PER_WORKER = BATCH // N_WORKERS
CHUNK = 128
N_CHUNKS = PER_WORKER // CHUNK
LANES = 16


def _sc_body(head_hbm, tail_hbm, table_hbm, w_hbm, b_hbm, out_hbm,
             hidx_v, tidx_v, hrows_v, trows_v, w_v, b_v, out_v, sem):
    wid = lax.axis_index("s") * NUM_CORES + lax.axis_index("c")
    base = wid * PER_WORKER

    pltpu.sync_copy(w_hbm, w_v)
    pltpu.sync_copy(b_hbm, b_v)
    bvec = b_v[...]
    wk = [w_v[pl.ds(k * LANES, LANES)] for k in range(EMBED_DIM // LANES)]
    lane_iota = jax.lax.iota(jnp.int32, LANES)
    lane_masks = [lane_iota == j for j in range(LANES)]

    def group_body(g, _):
        acc = jnp.zeros((LANES,), jnp.float32)
        for j in range(LANES):
            i = g * LANES + j
            dot = jnp.zeros((LANES,), jnp.float32)
            for k in range(EMBED_DIM // LANES):
                hv = hrows_v[i, pl.ds(k * LANES, LANES)]
                tv = trows_v[i, pl.ds(k * LANES, LANES)]
                dot = dot + hv * tv * wk[k]
            s = jnp.sum(dot)
            acc = jnp.where(lane_masks[j], s, acc)
        prob = 1.0 / (1.0 + jnp.exp(-(acc + bvec)))
        out_v[pl.ds(g * LANES, LANES)] = prob
        return _

    for c in range(N_CHUNKS):
        off = base + c * CHUNK
        pltpu.sync_copy(head_hbm.at[pl.ds(off, CHUNK)], hidx_v)
        pltpu.sync_copy(tail_hbm.at[pl.ds(off, CHUNK)], tidx_v)
        pltpu.async_copy(table_hbm.at[hidx_v], hrows_v, sem).wait()
        pltpu.async_copy(table_hbm.at[tidx_v], trows_v, sem).wait()
        lax.fori_loop(0, CHUNK // LANES, group_body, 0)
        pltpu.sync_copy(out_v, out_hbm.at[pl.ds(off, CHUNK)])


def kernel(head, tail, rel, embed_table, logreg_W, logreg_b):
    w = jnp.take(logreg_W, rel, axis=0).astype(jnp.float32)
    b = jnp.full((LANES,), jnp.take(logreg_b, rel), jnp.float32)

    mesh = plsc.VectorSubcoreMesh(core_axis_name="c", subcore_axis_name="s",
                                  num_cores=NUM_CORES, num_subcores=NUM_SUBCORES)
    run = pl.kernel(
        _sc_body,
        out_type=jax.ShapeDtypeStruct((BATCH,), jnp.float32),
        mesh=mesh,
        compiler_params=pltpu.CompilerParams(needs_layout_passes=False),
        scratch_types=[
            pltpu.VMEM((CHUNK,), jnp.int32),
            pltpu.VMEM((CHUNK,), jnp.int32),
            pltpu.VMEM((CHUNK, EMBED_DIM), jnp.float32),
            pltpu.VMEM((CHUNK, EMBED_DIM), jnp.float32),
            pltpu.VMEM((EMBED_DIM,), jnp.float32),
            pltpu.VMEM((LANES,), jnp.float32),
            pltpu.VMEM((CHUNK,), jnp.float32),
            pltpu.SemaphoreType.DMA,
        ],
    )
    return run(head, tail, embed_table, w, b)
